# baseline (speedup 1.0000x reference)
import functools

import jax
import jax.numpy as jnp
import numpy as np
from jax.experimental import pallas as pl
from jax.experimental.pallas import tpu as pltpu

_N = 32768
_N1 = 128
_N2 = 256
_K = 8
_R = 32


def _dft_constants():
    n1 = np.arange(_N1)
    n2 = np.arange(_N2)
    k2h = np.arange(_N1)
    f1 = np.exp(-2j * np.pi * np.outer(n1, n1) / _N1)
    tw = np.exp(-2j * np.pi * np.outer(n1, n2) / _N)
    tw = np.tile(tw, (1, _R))
    f2 = np.exp(-2j * np.pi * np.outer(n2, k2h) / _N2)
    asbf = lambda a: np.ascontiguousarray(a, dtype=np.float32).astype(
        jnp.bfloat16)
    return (asbf(f1.real), asbf(f1.imag), asbf(tw.real), asbf(tw.imag),
            asbf(f2.real), asbf(f2.imag))


_F1R, _F1I, _TWR, _TWI, _F2R, _F2I = _dft_constants()


def _bitonic_merge_desc(arr):
    n = len(arr)
    if n == 1:
        return arr
    half = n // 2
    top = [jnp.maximum(arr[i], arr[i + half]) for i in range(half)]
    bot = [jnp.minimum(arr[i], arr[i + half]) for i in range(half)]
    return _bitonic_merge_desc(top) + _bitonic_merge_desc(bot)


def _merge_desc(a, b):
    return _bitonic_merge_desc(a + b[::-1])


def _select_topk(dr, di):
    msq = dr * dr + di * di
    row_i = jax.lax.broadcasted_iota(jnp.int32, (_N1, _N1), 0)
    col_i = jax.lax.broadcasted_iota(jnp.int32, (_N1, _N1), 1)
    lin = row_i + _N1 * col_i
    key = jax.lax.bitcast_convert_type(msq, jnp.int32)
    key = jnp.bitwise_or(jnp.bitwise_and(key, -16384), 16383 - lin)
    key = jnp.where(lin == 0, 0, key)
    keyf = jax.lax.bitcast_convert_type(key, jnp.float32)
    w = [keyf[8 * j:8 * j + 8] for j in range(16)]
    runs = [_merge_desc([w[2 * i]], [w[2 * i + 1]]) for i in range(8)]
    runs = [_merge_desc(runs[2 * i], runs[2 * i + 1]) for i in range(4)]
    runs = [_merge_desc(runs[2 * i], runs[2 * i + 1]) for i in range(2)]
    a, b = runs
    s = _bitonic_merge_desc([jnp.maximum(a[i], b[7 - i]) for i in range(8)])
    for axis, levels in ((0, (1, 2, 4)), (1, (1, 2, 4, 8, 16, 32, 64))):
        for d in levels:
            rolled = [pltpu.roll(x, d, axis=axis) for x in s]
            s = _bitonic_merge_desc(
                [jnp.maximum(s[i], rolled[7 - i]) for i in range(8)])
    sub8 = jax.lax.broadcasted_iota(jnp.int32, (_K, _N1), 0)
    lane8 = jax.lax.broadcasted_iota(jnp.int32, (_K, _N1), 1)
    keysel = jnp.zeros((_K, _N1), jnp.int32)
    for j in range(_K):
        keysel = jnp.where(sub8 == j,
                           jax.lax.bitcast_convert_type(s[j], jnp.int32),
                           keysel)
    idx = 16383 - jnp.bitwise_and(keysel, 16383)
    rowi = jnp.bitwise_and(idx, 127)
    coli = jax.lax.shift_right_logical(idx, 7)
    rhot = (lane8 == rowi).astype(jnp.bfloat16)
    chot = (lane8 == coli).astype(jnp.float32)
    return keysel, rhot, chot


def _fft_topk_body(v_ref, f1r_ref, f1i_ref, twr_ref, twi_ref, f2r_ref,
                   f2i_ref, o_ref):
    dot = functools.partial(jnp.dot, preferred_element_type=jnp.float32)
    a = v_ref[...].astype(jnp.bfloat16)
    aw = jnp.concatenate([a[r * _N1:(r + 1) * _N1] for r in range(_R)],
                         axis=1)
    br = dot(f1r_ref[...], aw)
    bi = dot(f1i_ref[...], aw)
    brb = br.astype(jnp.bfloat16)
    bib = bi.astype(jnp.bfloat16)
    twr = twr_ref[...]
    twi = twi_ref[...]
    cr = brb * twr - bib * twi
    ci = brb * twi + bib * twr
    crv = jnp.concatenate([cr[:, r * _N2:(r + 1) * _N2] for r in range(_R)],
                          axis=0)
    civ = jnp.concatenate([ci[:, r * _N2:(r + 1) * _N2] for r in range(_R)],
                          axis=0)
    f2r = f2r_ref[...]
    f2i = f2i_ref[...]
    drv = dot(crv, f2r) - dot(civ, f2i)
    div = dot(crv, f2i) + dot(civ, f2r)
    sels = [_select_topk(drv[r * _N1:(r + 1) * _N1],
                         div[r * _N1:(r + 1) * _N1]) for r in range(_R)]
    drw = jnp.concatenate(
        [drv[r * _N1:(r + 1) * _N1] for r in range(_R)],
        axis=1).astype(jnp.bfloat16)
    diw = jnp.concatenate(
        [div[r * _N1:(r + 1) * _N1] for r in range(_R)],
        axis=1).astype(jnp.bfloat16)
    rh = jnp.concatenate([rhot for _, rhot, _ in sels], axis=0)
    mr = dot(rh, drw)
    mi = dot(rh, diw)
    res, ims, mgs = [], [], []
    for r in range(_R):
        keysel, _, chot = sels[r]
        blk = (slice(8 * r, 8 * r + 8), slice(_N1 * r, _N1 * (r + 1)))
        res.append(jnp.sum(mr[blk] * chot, axis=1, keepdims=True))
        ims.append(jnp.sum(mi[blk] * chot, axis=1, keepdims=True))
        msel = jnp.bitwise_or(jnp.bitwise_and(keysel, -16384), 8192)
        mgs.append(jnp.sqrt(
            jax.lax.bitcast_convert_type(msel, jnp.float32))[:, :1])
    rmat = jnp.concatenate(res, axis=1).T
    imat = jnp.concatenate(ims, axis=1).T
    mags = jnp.concatenate(mgs, axis=1).T
    phs = jnp.arctan2(imat, rmat)
    o_ref[...] = jnp.concatenate([mags, phs], axis=1)


def _fft_topk_call(v, rows, interpret=False):
    const_spec = lambda shape: pl.BlockSpec(shape, lambda i: (0, 0))
    return pl.pallas_call(
        _fft_topk_body,
        grid=(rows // _R,),
        in_specs=[
            pl.BlockSpec((_R * _N1, _N2), lambda i: (i, 0)),
            const_spec((_N1, _N1)),
            const_spec((_N1, _N1)),
            const_spec((_N1, _R * _N2)),
            const_spec((_N1, _R * _N2)),
            const_spec((_N2, _N1)),
            const_spec((_N2, _N1)),
        ],
        out_specs=pl.BlockSpec((_R, 2 * _K), lambda i: (i, 0)),
        out_shape=jax.ShapeDtypeStruct((rows, 2 * _K), jnp.float32),
        compiler_params=pltpu.CompilerParams(
            dimension_semantics=("parallel",)),
        interpret=interpret,
    )(v, _F1R, _F1I, _TWR, _TWI, _F2R, _F2I)


def kernel(x):
    rows = x.shape[0]
    v = x.reshape(rows * _N1, _N2)
    return _fft_topk_call(v, rows)


# speedup vs baseline: 1.1699x; 1.1699x over previous
import functools

import jax
import jax.numpy as jnp
import numpy as np
from jax.experimental import pallas as pl
from jax.experimental.pallas import tpu as pltpu

_N = 32768
_N1 = 128
_N2 = 256
_K = 8
_R = 16


def _dft_constants():
    n1 = np.arange(_N1)
    n2 = np.arange(_N2)
    k2h = np.arange(_N1)
    f1 = np.exp(-2j * np.pi * np.outer(n1, n1) / _N1)
    tw = np.exp(-2j * np.pi * np.outer(n1, n2) / _N)
    tw = np.tile(tw, (1, _R))
    f2 = np.exp(-2j * np.pi * np.outer(n2, k2h) / _N2)
    asbf = lambda a: np.ascontiguousarray(a, dtype=np.float32).astype(
        jnp.bfloat16)
    return (asbf(f1.real), asbf(f1.imag), asbf(tw.real), asbf(tw.imag),
            asbf(f2.real), asbf(f2.imag))


_F1R, _F1I, _TWR, _TWI, _F2R, _F2I = _dft_constants()


def _bitonic_merge_desc(arr):
    n = len(arr)
    if n == 1:
        return arr
    half = n // 2
    top = [jnp.maximum(arr[i], arr[i + half]) for i in range(half)]
    bot = [jnp.minimum(arr[i], arr[i + half]) for i in range(half)]
    return _bitonic_merge_desc(top) + _bitonic_merge_desc(bot)


def _merge_desc(a, b):
    return _bitonic_merge_desc(a + b[::-1])


def _select_topk(dr, di):
    msq = dr * dr + di * di
    row_i = jax.lax.broadcasted_iota(jnp.int32, (_N1, _N1), 0)
    col_i = jax.lax.broadcasted_iota(jnp.int32, (_N1, _N1), 1)
    lin = row_i + _N1 * col_i
    key = jax.lax.bitcast_convert_type(msq, jnp.int32)
    key = jnp.bitwise_or(jnp.bitwise_and(key, -16384), 16383 - lin)
    key = jnp.where(lin == 0, 0, key)
    keyf = jax.lax.bitcast_convert_type(key, jnp.float32)
    w = [keyf[8 * j:8 * j + 8] for j in range(16)]
    runs = [_merge_desc([w[2 * i]], [w[2 * i + 1]]) for i in range(8)]
    runs = [_merge_desc(runs[2 * i], runs[2 * i + 1]) for i in range(4)]
    runs = [_merge_desc(runs[2 * i], runs[2 * i + 1]) for i in range(2)]
    a, b = runs
    s = _bitonic_merge_desc([jnp.maximum(a[i], b[7 - i]) for i in range(8)])
    for axis, levels in ((0, (1, 2, 4)), (1, (1, 2, 4, 8, 16, 32, 64))):
        for d in levels:
            rolled = [pltpu.roll(x, d, axis=axis) for x in s]
            s = _bitonic_merge_desc(
                [jnp.maximum(s[i], rolled[7 - i]) for i in range(8)])
    sub8 = jax.lax.broadcasted_iota(jnp.int32, (_K, _N1), 0)
    lane8 = jax.lax.broadcasted_iota(jnp.int32, (_K, _N1), 1)
    keysel = jnp.zeros((_K, _N1), jnp.int32)
    for j in range(_K):
        keysel = jnp.where(sub8 == j,
                           jax.lax.bitcast_convert_type(s[j], jnp.int32),
                           keysel)
    idx = 16383 - jnp.bitwise_and(keysel, 16383)
    rowi = jnp.bitwise_and(idx, 127)
    coli = jax.lax.shift_right_logical(idx, 7)
    rhot = (lane8 == rowi).astype(jnp.bfloat16)
    chot = (lane8 == coli).astype(jnp.float32)
    return keysel, rhot, chot


def _fft_topk_body(v_ref, f1r_ref, f1i_ref, twr_ref, twi_ref, f2r_ref,
                   f2i_ref, o_ref):
    dot = functools.partial(jnp.dot, preferred_element_type=jnp.float32)
    a = v_ref[...].astype(jnp.bfloat16)
    aw = jnp.concatenate([a[r * _N1:(r + 1) * _N1] for r in range(_R)],
                         axis=1)
    br = dot(f1r_ref[...], aw)
    bi = dot(f1i_ref[...], aw)
    brb = br.astype(jnp.bfloat16)
    bib = bi.astype(jnp.bfloat16)
    twr = twr_ref[...]
    twi = twi_ref[...]
    cr = brb * twr - bib * twi
    ci = brb * twi + bib * twr
    crv = jnp.concatenate([cr[:, r * _N2:(r + 1) * _N2] for r in range(_R)],
                          axis=0)
    civ = jnp.concatenate([ci[:, r * _N2:(r + 1) * _N2] for r in range(_R)],
                          axis=0)
    f2r = f2r_ref[...]
    f2i = f2i_ref[...]
    drv = dot(crv, f2r) - dot(civ, f2i)
    div = dot(crv, f2i) + dot(civ, f2r)
    sels = [_select_topk(drv[r * _N1:(r + 1) * _N1],
                         div[r * _N1:(r + 1) * _N1]) for r in range(_R)]
    res, ims, mgs = [], [], []
    grp = 4
    for g in range(_R // grp):
        rows_g = range(g * grp, (g + 1) * grp)
        drw = jnp.concatenate(
            [drv[r * _N1:(r + 1) * _N1] for r in rows_g],
            axis=1).astype(jnp.bfloat16)
        diw = jnp.concatenate(
            [div[r * _N1:(r + 1) * _N1] for r in rows_g],
            axis=1).astype(jnp.bfloat16)
        rh = jnp.concatenate([sels[r][1] for r in rows_g], axis=0)
        mr = dot(rh, drw)
        mi = dot(rh, diw)
        for j, r in enumerate(rows_g):
            keysel, _, chot = sels[r]
            blk = (slice(8 * j, 8 * j + 8), slice(_N1 * j, _N1 * (j + 1)))
            res.append(jnp.sum(mr[blk] * chot, axis=1, keepdims=True))
            ims.append(jnp.sum(mi[blk] * chot, axis=1, keepdims=True))
            msel = jnp.bitwise_or(jnp.bitwise_and(keysel, -16384), 8192)
            mgs.append(jnp.sqrt(
                jax.lax.bitcast_convert_type(msel, jnp.float32))[:, :1])
    rmat = jnp.concatenate(res, axis=1).T
    imat = jnp.concatenate(ims, axis=1).T
    mags = jnp.concatenate(mgs, axis=1).T
    phs = jnp.arctan2(imat, rmat)
    o_ref[...] = jnp.concatenate([mags, phs], axis=1)


def _fft_topk_call(v, rows, interpret=False):
    const_spec = lambda shape: pl.BlockSpec(shape, lambda i: (0, 0))
    return pl.pallas_call(
        _fft_topk_body,
        grid=(rows // _R,),
        in_specs=[
            pl.BlockSpec((_R * _N1, _N2), lambda i: (i, 0)),
            const_spec((_N1, _N1)),
            const_spec((_N1, _N1)),
            const_spec((_N1, _R * _N2)),
            const_spec((_N1, _R * _N2)),
            const_spec((_N2, _N1)),
            const_spec((_N2, _N1)),
        ],
        out_specs=pl.BlockSpec((_R, 2 * _K), lambda i: (i, 0)),
        out_shape=jax.ShapeDtypeStruct((rows, 2 * _K), jnp.float32),
        compiler_params=pltpu.CompilerParams(
            dimension_semantics=("parallel",)),
        interpret=interpret,
    )(v, _F1R, _F1I, _TWR, _TWI, _F2R, _F2I)


def kernel(x):
    rows = x.shape[0]
    v = x.reshape(rows * _N1, _N2)
    return _fft_topk_call(v, rows)


# speedup vs baseline: 1.1744x; 1.0039x over previous
import functools

import jax
import jax.numpy as jnp
import numpy as np
from jax.experimental import pallas as pl
from jax.experimental.pallas import tpu as pltpu

_N = 32768
_N1 = 128
_N2 = 256
_K = 8
_R = 16


def _dft_constants():
    n1 = np.arange(_N1)
    n2 = np.arange(_N2)
    k2h = np.arange(_N1)
    f1 = np.exp(-2j * np.pi * np.outer(n1, n1) / _N1)
    tw = np.exp(-2j * np.pi * np.outer(n1, n2) / _N)
    tw = np.tile(tw, (1, _R))
    f2 = np.exp(-2j * np.pi * np.outer(n2, k2h) / _N2)
    asbf = lambda a: np.ascontiguousarray(a, dtype=np.float32).astype(
        jnp.bfloat16)
    return (asbf(f1.real), asbf(f1.imag), asbf(tw.real), asbf(tw.imag),
            asbf(f2.real), asbf(f2.imag))


_F1R, _F1I, _TWR, _TWI, _F2R, _F2I = _dft_constants()


def _bitonic_merge_desc(arr):
    n = len(arr)
    if n == 1:
        return arr
    half = n // 2
    top = [jnp.maximum(arr[i], arr[i + half]) for i in range(half)]
    bot = [jnp.minimum(arr[i], arr[i + half]) for i in range(half)]
    return _bitonic_merge_desc(top) + _bitonic_merge_desc(bot)


def _merge_desc(a, b):
    return _bitonic_merge_desc(a + b[::-1])


def _select_topk(dr, di):
    msq = dr * dr + di * di
    row_i = jax.lax.broadcasted_iota(jnp.int32, (_N1, _N1), 0)
    col_i = jax.lax.broadcasted_iota(jnp.int32, (_N1, _N1), 1)
    lin = row_i + _N1 * col_i
    key = jax.lax.bitcast_convert_type(msq, jnp.int32)
    key = jnp.bitwise_or(jnp.bitwise_and(key, -16384), 16383 - lin)
    key = jnp.where(lin == 0, 0, key)
    keyf = jax.lax.bitcast_convert_type(key, jnp.float32)
    w = [keyf[8 * j:8 * j + 8] for j in range(16)]
    runs = [_merge_desc([w[2 * i]], [w[2 * i + 1]]) for i in range(8)]
    runs = [_merge_desc(runs[2 * i], runs[2 * i + 1]) for i in range(4)]
    runs = [_merge_desc(runs[2 * i], runs[2 * i + 1]) for i in range(2)]
    a, b = runs
    s = _bitonic_merge_desc([jnp.maximum(a[i], b[7 - i]) for i in range(8)])
    for axis, levels in ((0, (1, 2, 4)), (1, (1, 2, 4, 8, 16, 32, 64))):
        for d in levels:
            rolled = [pltpu.roll(x, d, axis=axis) for x in s]
            s = _bitonic_merge_desc(
                [jnp.maximum(s[i], rolled[7 - i]) for i in range(8)])
    sub8 = jax.lax.broadcasted_iota(jnp.int32, (_K, _N1), 0)
    lane8 = jax.lax.broadcasted_iota(jnp.int32, (_K, _N1), 1)
    keysel = jnp.zeros((_K, _N1), jnp.int32)
    for j in range(_K):
        keysel = jnp.where(sub8 == j,
                           jax.lax.bitcast_convert_type(s[j], jnp.int32),
                           keysel)
    idx = 16383 - jnp.bitwise_and(keysel, 16383)
    rowi = jnp.bitwise_and(idx, 127)
    coli = jax.lax.shift_right_logical(idx, 7)
    rhot = (lane8 == rowi).astype(jnp.bfloat16)
    chot = (lane8 == coli).astype(jnp.float32)
    return keysel, rhot, chot


def _fft_topk_body(v_ref, f1r_ref, f1i_ref, twr_ref, twi_ref, f2r_ref,
                   f2i_ref, o_ref):
    dot = functools.partial(jnp.dot, preferred_element_type=jnp.float32)
    a = v_ref[...].astype(jnp.bfloat16)
    aw = jnp.concatenate([a[r * _N1:(r + 1) * _N1] for r in range(_R)],
                         axis=1)
    br = dot(f1r_ref[...], aw)
    bi = dot(f1i_ref[...], aw)
    brb = br.astype(jnp.bfloat16)
    bib = bi.astype(jnp.bfloat16)
    twr = twr_ref[...]
    twi = twi_ref[...]
    cr = brb * twr - bib * twi
    ci = brb * twi + bib * twr
    crv = jnp.concatenate([cr[:, r * _N2:(r + 1) * _N2] for r in range(_R)],
                          axis=0)
    civ = jnp.concatenate([ci[:, r * _N2:(r + 1) * _N2] for r in range(_R)],
                          axis=0)
    f2r = f2r_ref[...]
    f2i = f2i_ref[...]
    drv = dot(crv, f2r) - dot(civ, f2i)
    div = dot(crv, f2i) + dot(civ, f2r)
    sels = [_select_topk(drv[r * _N1:(r + 1) * _N1],
                         div[r * _N1:(r + 1) * _N1]) for r in range(_R)]
    res, ims, mgs = [], [], []
    grp = 2
    for g in range(_R // grp):
        rows_g = range(g * grp, (g + 1) * grp)
        drw = jnp.concatenate(
            [drv[r * _N1:(r + 1) * _N1] for r in rows_g],
            axis=1).astype(jnp.bfloat16)
        diw = jnp.concatenate(
            [div[r * _N1:(r + 1) * _N1] for r in rows_g],
            axis=1).astype(jnp.bfloat16)
        rh = jnp.concatenate([sels[r][1] for r in rows_g], axis=0)
        mr = dot(rh, drw)
        mi = dot(rh, diw)
        for j, r in enumerate(rows_g):
            keysel, _, chot = sels[r]
            blk = (slice(8 * j, 8 * j + 8), slice(_N1 * j, _N1 * (j + 1)))
            res.append(jnp.sum(mr[blk] * chot, axis=1, keepdims=True))
            ims.append(jnp.sum(mi[blk] * chot, axis=1, keepdims=True))
            msel = jnp.bitwise_or(jnp.bitwise_and(keysel, -16384), 8192)
            mgs.append(jnp.sqrt(
                jax.lax.bitcast_convert_type(msel, jnp.float32))[:, :1])
    rmat = jnp.concatenate(res, axis=1).T
    imat = jnp.concatenate(ims, axis=1).T
    mags = jnp.concatenate(mgs, axis=1).T
    phs = jnp.arctan2(imat, rmat)
    o_ref[...] = jnp.concatenate([mags, phs], axis=1)


def _fft_topk_call(v, rows, interpret=False):
    const_spec = lambda shape: pl.BlockSpec(shape, lambda i: (0, 0))
    return pl.pallas_call(
        _fft_topk_body,
        grid=(rows // _R,),
        in_specs=[
            pl.BlockSpec((_R * _N1, _N2), lambda i: (i, 0)),
            const_spec((_N1, _N1)),
            const_spec((_N1, _N1)),
            const_spec((_N1, _R * _N2)),
            const_spec((_N1, _R * _N2)),
            const_spec((_N2, _N1)),
            const_spec((_N2, _N1)),
        ],
        out_specs=pl.BlockSpec((_R, 2 * _K), lambda i: (i, 0)),
        out_shape=jax.ShapeDtypeStruct((rows, 2 * _K), jnp.float32),
        compiler_params=pltpu.CompilerParams(
            dimension_semantics=("parallel",)),
        interpret=interpret,
    )(v, _F1R, _F1I, _TWR, _TWI, _F2R, _F2I)


def kernel(x):
    rows = x.shape[0]
    v = x.reshape(rows * _N1, _N2)
    return _fft_topk_call(v, rows)
